# Initial kernel scaffold; baseline (speedup 1.0000x reference)
import jax
import jax.numpy as jnp
from jax.experimental import pallas as pl


def kernel(input, weights):
    raise NotImplementedError("write your pallas kernel here")


# speedup vs baseline: 2.0034x; 2.0034x over previous
import functools

import jax
import jax.numpy as jnp
from jax import lax
from jax.experimental import pallas as pl
from jax.experimental.pallas import tpu as pltpu
from jax.experimental.pallas import tpu_sc as plsc

PADDING_IDX = 0
CHUNK = 32


@functools.lru_cache(maxsize=None)
def _build(ntok, emb):
    info = plsc.get_sparse_core_info()
    nc, ns, lanes = info.num_cores, info.num_subcores, info.num_lanes
    nw = nc * ns
    per_w = ntok // nw
    assert ntok % nw == 0 and per_w % CHUNK == 0 and per_w % lanes == 0
    nchunks = per_w // CHUNK
    mesh = plsc.VectorSubcoreMesh(core_axis_name="c", subcore_axis_name="s")

    @functools.partial(
        pl.kernel,
        mesh=mesh,
        out_type=jax.ShapeDtypeStruct((ntok, emb), jnp.float32),
        scratch_types=[
            pltpu.VMEM((per_w,), jnp.int32),
            pltpu.VMEM((per_w,), jnp.int32),
            pltpu.VMEM((CHUNK, emb), jnp.float32),
            pltpu.SemaphoreType.DMA,
        ],
    )
    def k(inp_hbm, w_hbm, out_hbm, tok_v, idx_v, rows_v, sem):
        wid = lax.axis_index("s") * nc + lax.axis_index("c")
        base = wid * per_w
        pltpu.sync_copy(inp_hbm.at[pl.ds(base, per_w)], tok_v)
        for i in range(per_w // lanes):
            t = tok_v[pl.ds(i * lanes, lanes)]
            pos = lax.iota(jnp.int32, lanes) + (base + i * lanes + 1)
            idx_v[pl.ds(i * lanes, lanes)] = jnp.where(
                t != PADDING_IDX, pos, PADDING_IDX)

        def body(g, carry):
            t0 = pl.multiple_of(g * CHUNK, CHUNK)
            pltpu.async_copy(w_hbm.at[idx_v.at[pl.ds(t0, CHUNK)]],
                             rows_v, sem).wait()
            pltpu.sync_copy(rows_v, out_hbm.at[pl.ds(base + t0, CHUNK)])
            return carry

        lax.fori_loop(0, nchunks, body, 0)

    return k


def kernel(input, weights):
    bsz, seq_len = input.shape
    emb = weights.shape[1]
    ntok = bsz * seq_len
    out = _build(ntok, emb)(input.reshape(ntok), weights)
    return out.reshape(bsz, seq_len, emb)


# speedup vs baseline: 2.2435x; 1.1198x over previous
import functools

import jax
import jax.numpy as jnp
from jax import lax
from jax.experimental import pallas as pl
from jax.experimental.pallas import tpu as pltpu
from jax.experimental.pallas import tpu_sc as plsc

PADDING_IDX = 0
CHUNK = 32


@functools.lru_cache(maxsize=None)
def _build(ntok, emb):
    info = plsc.get_sparse_core_info()
    nc, ns, lanes = info.num_cores, info.num_subcores, info.num_lanes
    nw = nc * ns
    per_w = ntok // nw
    assert ntok % nw == 0 and per_w % CHUNK == 0 and per_w % lanes == 0
    nchunks = per_w // CHUNK
    mesh = plsc.VectorSubcoreMesh(core_axis_name="c", subcore_axis_name="s")

    npairs = nchunks // 2
    assert nchunks % 2 == 0

    @functools.partial(
        pl.kernel,
        mesh=mesh,
        out_type=jax.ShapeDtypeStruct((ntok, emb), jnp.float32),
        scratch_types=[
            pltpu.VMEM((per_w,), jnp.int32),
            pltpu.VMEM((per_w,), jnp.int32),
            pltpu.VMEM((CHUNK, emb), jnp.float32),
            pltpu.VMEM((CHUNK, emb), jnp.float32),
            pltpu.SemaphoreType.DMA,
            pltpu.SemaphoreType.DMA,
            pltpu.SemaphoreType.DMA,
            pltpu.SemaphoreType.DMA,
        ],
    )
    def k(inp_hbm, w_hbm, out_hbm, tok_v, idx_v, rows0, rows1,
          gs0, gs1, ss0, ss1):
        wid = lax.axis_index("s") * nc + lax.axis_index("c")
        base = wid * per_w
        pltpu.sync_copy(inp_hbm.at[pl.ds(base, per_w)], tok_v)
        for i in range(per_w // lanes):
            t = tok_v[pl.ds(i * lanes, lanes)]
            pos = lax.iota(jnp.int32, lanes) + (base + i * lanes + 1)
            idx_v[pl.ds(i * lanes, lanes)] = jnp.where(
                t != PADDING_IDX, pos, PADDING_IDX)

        bufs = ((rows0, gs0, ss0), (rows1, gs1, ss1))

        def start_gather(c, buf, gsem):
            t0 = pl.multiple_of(c * CHUNK, CHUNK)
            pltpu.async_copy(w_hbm.at[idx_v.at[pl.ds(t0, CHUNK)]], buf, gsem)

        def wait_gather(buf, gsem):
            pltpu.make_async_copy(w_hbm.at[pl.ds(0, CHUNK)], buf, gsem).wait()

        def start_scatter(c, buf, ssem):
            t0 = pl.multiple_of(base + c * CHUNK, CHUNK)
            pltpu.async_copy(buf, out_hbm.at[pl.ds(t0, CHUNK)], ssem)

        def wait_scatter(buf, ssem):
            pltpu.make_async_copy(
                buf, out_hbm.at[pl.ds(base, CHUNK)], ssem).wait()

        start_gather(0, rows0, gs0)
        start_gather(1, rows1, gs1)

        def body(p, carry):
            for j, (buf, gsem, ssem) in enumerate(bufs):
                c = 2 * p + j
                wait_gather(buf, gsem)
                start_scatter(c, buf, ssem)
            @pl.when(p + 1 < npairs)
            def _():
                for j, (buf, gsem, ssem) in enumerate(bufs):
                    wait_scatter(buf, ssem)
                    start_gather(2 * p + 2 + j, buf, gsem)
            return carry

        lax.fori_loop(0, npairs, body, 0)
        wait_scatter(rows0, ss0)
        wait_scatter(rows1, ss1)

    return k


def kernel(input, weights):
    bsz, seq_len = input.shape
    emb = weights.shape[1]
    ntok = bsz * seq_len
    out = _build(ntok, emb)(input.reshape(ntok), weights)
    return out.reshape(bsz, seq_len, emb)


# speedup vs baseline: 2.3334x; 1.0400x over previous
import functools

import jax
import jax.numpy as jnp
from jax import lax
from jax.experimental import pallas as pl
from jax.experimental.pallas import tpu as pltpu
from jax.experimental.pallas import tpu_sc as plsc

PADDING_IDX = 0
CHUNK = 16
NBUF = 4


@functools.lru_cache(maxsize=None)
def _build(ntok, emb):
    info = plsc.get_sparse_core_info()
    nc, ns, lanes = info.num_cores, info.num_subcores, info.num_lanes
    nw = nc * ns
    per_w = ntok // nw
    assert ntok % nw == 0 and per_w % CHUNK == 0 and per_w % lanes == 0
    nchunks = per_w // CHUNK
    assert nchunks % NBUF == 0
    ngroups = nchunks // NBUF
    mesh = plsc.VectorSubcoreMesh(core_axis_name="c", subcore_axis_name="s")

    scratch = [
        pltpu.VMEM((per_w,), jnp.int32),
        pltpu.VMEM((per_w,), jnp.int32),
    ]
    scratch += [pltpu.VMEM((CHUNK, emb), jnp.float32) for _ in range(NBUF)]
    scratch += [pltpu.SemaphoreType.DMA for _ in range(2 * NBUF)]

    @functools.partial(
        pl.kernel,
        mesh=mesh,
        out_type=jax.ShapeDtypeStruct((ntok, emb), jnp.float32),
        scratch_types=scratch,
    )
    def k(inp_hbm, w_hbm, out_hbm, tok_v, idx_v, *bufs_sems):
        rows = bufs_sems[:NBUF]
        gsems = bufs_sems[NBUF:2 * NBUF]
        ssems = bufs_sems[2 * NBUF:]
        wid = lax.axis_index("s") * nc + lax.axis_index("c")
        base = wid * per_w
        pltpu.sync_copy(inp_hbm.at[pl.ds(base, per_w)], tok_v)
        for i in range(per_w // lanes):
            t = tok_v[pl.ds(i * lanes, lanes)]
            pos = lax.iota(jnp.int32, lanes) + (base + i * lanes + 1)
            idx_v[pl.ds(i * lanes, lanes)] = jnp.where(
                t != PADDING_IDX, pos, PADDING_IDX)

        def start_gather(c, buf, gsem):
            t0 = pl.multiple_of(c * CHUNK, CHUNK)
            pltpu.async_copy(w_hbm.at[idx_v.at[pl.ds(t0, CHUNK)]], buf, gsem)

        def wait_gather(buf, gsem):
            pltpu.make_async_copy(w_hbm.at[pl.ds(0, CHUNK)], buf, gsem).wait()

        def start_scatter(c, buf, ssem):
            t0 = pl.multiple_of(base + c * CHUNK, CHUNK)
            pltpu.async_copy(buf, out_hbm.at[pl.ds(t0, CHUNK)], ssem)

        def wait_scatter(buf, ssem):
            pltpu.make_async_copy(
                buf, out_hbm.at[pl.ds(base, CHUNK)], ssem).wait()

        for j in range(NBUF):
            start_gather(j, rows[j], gsems[j])

        def body(p, carry):
            for j in range(NBUF):
                c = NBUF * p + j
                wait_gather(rows[j], gsems[j])
                start_scatter(c, rows[j], ssems[j])
            @pl.when(p + 1 < ngroups)
            def _():
                for j in range(NBUF):
                    wait_scatter(rows[j], ssems[j])
                    start_gather(NBUF * (p + 1) + j, rows[j], gsems[j])
            return carry

        lax.fori_loop(0, ngroups, body, 0)
        for j in range(NBUF):
            wait_scatter(rows[j], ssems[j])

    return k


def kernel(input, weights):
    bsz, seq_len = input.shape
    emb = weights.shape[1]
    ntok = bsz * seq_len
    out = _build(ntok, emb)(input.reshape(ntok), weights)
    return out.reshape(bsz, seq_len, emb)


# speedup vs baseline: 3.0812x; 1.3205x over previous
import functools

import jax
import jax.numpy as jnp
from jax import lax
from jax.experimental import pallas as pl
from jax.experimental.pallas import tpu as pltpu
from jax.experimental.pallas import tpu_sc as plsc

PADDING_IDX = 0
CHUNK = 16
NBUF = 4


@functools.lru_cache(maxsize=None)
def _build(bsz, seq_len, emb):
    info = plsc.get_sparse_core_info()
    nc, ns, lanes = info.num_cores, info.num_subcores, info.num_lanes
    nw = nc * ns
    ntok = bsz * seq_len
    per_w = seq_len // nw
    assert seq_len % nw == 0 and per_w % CHUNK == 0 and CHUNK == lanes
    nchunks = per_w // CHUNK
    assert nchunks % NBUF == 0
    ngroups = nchunks // NBUF
    mesh = plsc.VectorSubcoreMesh(core_axis_name="c", subcore_axis_name="s")

    scratch = [
        pltpu.VMEM((bsz * per_w,), jnp.int32),
        pltpu.VMEM((per_w,), jnp.int32),
        pltpu.VMEM((lanes, emb), jnp.float32),
    ]
    scratch += [pltpu.VMEM((CHUNK, emb), jnp.float32) for _ in range(NBUF)]
    scratch += [pltpu.SemaphoreType.DMA for _ in range(2 * NBUF + 1)]

    @functools.partial(
        pl.kernel,
        mesh=mesh,
        out_type=jax.ShapeDtypeStruct((ntok, emb), jnp.float32),
        scratch_types=scratch,
    )
    def k(inp_hbm, w_hbm, pos_hbm, out_hbm, tok_v, ridx_v, frows_v, *rest):
        rows = rest[:NBUF]
        gsems = rest[NBUF:2 * NBUF]
        ssems = rest[2 * NBUF:3 * NBUF]
        fsem = rest[3 * NBUF]
        wid = lax.axis_index("s") * nc + lax.axis_index("c")
        s0 = wid * per_w

        for b in range(bsz):
            pltpu.sync_copy(inp_hbm.at[pl.ds(b * seq_len + s0, per_w)],
                            tok_v.at[pl.ds(b * per_w, per_w)])
        pltpu.sync_copy(pos_hbm.at[pl.ds(s0, per_w)], ridx_v)

        lane_iota = lax.iota(jnp.int32, lanes)

        def start_gather(c, buf, gsem):
            t0 = pl.multiple_of(c * CHUNK, CHUNK)
            pltpu.async_copy(w_hbm.at[ridx_v.at[pl.ds(t0, CHUNK)]], buf, gsem)

        def wait_gather(buf, gsem):
            pltpu.make_async_copy(w_hbm.at[pl.ds(0, CHUNK)], buf, gsem).wait()

        def start_scatters(c, buf, ssem):
            for b in range(bsz):
                t0 = b * seq_len + s0 + c * CHUNK
                pltpu.async_copy(buf, out_hbm.at[pl.ds(t0, CHUNK)], ssem)

        def wait_scatters(buf, ssem):
            for _ in range(bsz):
                pltpu.make_async_copy(
                    buf, out_hbm.at[pl.ds(s0, CHUNK)], ssem).wait()

        for j in range(NBUF):
            start_gather(j, rows[j], gsems[j])

        def body(p, carry):
            for j in range(NBUF):
                c = NBUF * p + j
                wait_gather(rows[j], gsems[j])
                start_scatters(c, rows[j], ssems[j])
            @pl.when(p + 1 < ngroups)
            def _():
                for j in range(NBUF):
                    wait_scatters(rows[j], ssems[j])
                    start_gather(NBUF * (p + 1) + j, rows[j], gsems[j])
            return carry

        lax.fori_loop(0, ngroups, body, 0)
        for j in range(NBUF):
            wait_scatters(rows[j], ssems[j])

        def lane_min(v):
            for s in (1, 2, 4, 8):
                perm = jnp.bitwise_xor(lane_iota, s)
                shuf = lax.gather(
                    v, perm[:, None],
                    lax.GatherDimensionNumbers(
                        offset_dims=(), collapsed_slice_dims=(0,),
                        start_index_map=(0,)),
                    (1,), mode=lax.GatherScatterMode.PROMISE_IN_BOUNDS)
                v = jnp.minimum(v, shuf)
            return v

        def fix_group(b, g, carry):
            t = tok_v[pl.ds(b * per_w + g * lanes, lanes)]
            mask = t == PADDING_IDX
            has_pad = lane_min(t)[0] == PADDING_IDX
            trows = lane_iota + (s0 + g * lanes + 1)
            @pl.when(has_pad)
            def _():
                fidx = jnp.where(mask, PADDING_IDX, trows)
                pltpu.async_copy(w_hbm.at[fidx], frows_v, fsem).wait()
                pltpu.async_copy(
                    frows_v,
                    out_hbm.at[pl.ds(b * seq_len + s0 + g * lanes, lanes)],
                    fsem).wait()
            return carry

        def fix_batch(b, carry):
            return lax.fori_loop(
                0, per_w // lanes, functools.partial(fix_group, b), carry)

        lax.fori_loop(0, bsz, fix_batch, 0)

    return k


def kernel(input, weights):
    bsz, seq_len = input.shape
    emb = weights.shape[1]
    pos = jnp.arange(1, seq_len + 1, dtype=jnp.int32)
    out = _build(bsz, seq_len, emb)(input.reshape(-1), weights, pos)
    return out.reshape(bsz, seq_len, emb)


# speedup vs baseline: 3.2603x; 1.0581x over previous
import functools

import jax
import jax.numpy as jnp
from jax import lax
from jax.experimental import pallas as pl
from jax.experimental.pallas import tpu as pltpu
from jax.experimental.pallas import tpu_sc as plsc

PADDING_IDX = 0
CHUNK = 32
NBUF = 2


@functools.lru_cache(maxsize=None)
def _build(bsz, seq_len, emb):
    info = plsc.get_sparse_core_info()
    nc, ns, lanes = info.num_cores, info.num_subcores, info.num_lanes
    nw = nc * ns
    ntok = bsz * seq_len
    per_w = seq_len // nw
    assert seq_len % nw == 0 and per_w % CHUNK == 0 and CHUNK % lanes == 0
    nchunks = per_w // CHUNK
    assert nchunks % NBUF == 0
    ngroups = nchunks // NBUF
    mesh = plsc.VectorSubcoreMesh(core_axis_name="c", subcore_axis_name="s")

    scratch = [
        pltpu.VMEM((bsz * per_w,), jnp.int32),
        pltpu.VMEM((per_w,), jnp.int32),
        pltpu.VMEM((lanes, emb), jnp.float32),
    ]
    scratch += [pltpu.VMEM((CHUNK, emb), jnp.float32) for _ in range(NBUF)]
    scratch += [pltpu.SemaphoreType.DMA for _ in range(2 * NBUF + 1)]

    @functools.partial(
        pl.kernel,
        mesh=mesh,
        out_type=jax.ShapeDtypeStruct((ntok, emb), jnp.float32),
        scratch_types=scratch,
    )
    def k(inp_hbm, w_hbm, pos_hbm, out_hbm, tok_v, ridx_v, frows_v, *rest):
        rows = rest[:NBUF]
        gsems = rest[NBUF:2 * NBUF]
        ssems = rest[2 * NBUF:3 * NBUF]
        fsem = rest[3 * NBUF]
        wid = lax.axis_index("s") * nc + lax.axis_index("c")
        s0 = wid * per_w

        for b in range(bsz):
            pltpu.sync_copy(inp_hbm.at[pl.ds(b * seq_len + s0, per_w)],
                            tok_v.at[pl.ds(b * per_w, per_w)])
        pltpu.sync_copy(pos_hbm.at[pl.ds(s0, per_w)], ridx_v)

        lane_iota = lax.iota(jnp.int32, lanes)

        def start_gather(c, buf, gsem):
            t0 = pl.multiple_of(c * CHUNK, CHUNK)
            pltpu.async_copy(w_hbm.at[ridx_v.at[pl.ds(t0, CHUNK)]], buf, gsem)

        def wait_gather(buf, gsem):
            pltpu.make_async_copy(w_hbm.at[pl.ds(0, CHUNK)], buf, gsem).wait()

        def start_scatters(c, buf, ssem):
            for b in range(bsz):
                t0 = b * seq_len + s0 + c * CHUNK
                pltpu.async_copy(buf, out_hbm.at[pl.ds(t0, CHUNK)], ssem)

        def wait_scatters(buf, ssem):
            for _ in range(bsz):
                pltpu.make_async_copy(
                    buf, out_hbm.at[pl.ds(s0, CHUNK)], ssem).wait()

        for j in range(NBUF):
            start_gather(j, rows[j], gsems[j])

        def body(p, carry):
            for j in range(NBUF):
                c = NBUF * p + j
                wait_gather(rows[j], gsems[j])
                start_scatters(c, rows[j], ssems[j])
            @pl.when(p + 1 < ngroups)
            def _():
                for j in range(NBUF):
                    wait_scatters(rows[j], ssems[j])
                    start_gather(NBUF * (p + 1) + j, rows[j], gsems[j])
            return carry

        lax.fori_loop(0, ngroups, body, 0)
        for j in range(NBUF):
            wait_scatters(rows[j], ssems[j])

        def lane_min(v):
            for s in (1, 2, 4, 8):
                perm = jnp.bitwise_xor(lane_iota, s)
                shuf = lax.gather(
                    v, perm[:, None],
                    lax.GatherDimensionNumbers(
                        offset_dims=(), collapsed_slice_dims=(0,),
                        start_index_map=(0,)),
                    (1,), mode=lax.GatherScatterMode.PROMISE_IN_BOUNDS)
                v = jnp.minimum(v, shuf)
            return v

        def fix_group(b, g, carry):
            t = tok_v[pl.ds(b * per_w + g * lanes, lanes)]
            mask = t == PADDING_IDX
            has_pad = lane_min(t)[0] == PADDING_IDX
            trows = lane_iota + (s0 + g * lanes + 1)
            @pl.when(has_pad)
            def _():
                fidx = jnp.where(mask, PADDING_IDX, trows)
                pltpu.async_copy(w_hbm.at[fidx], frows_v, fsem).wait()
                pltpu.async_copy(
                    frows_v,
                    out_hbm.at[pl.ds(b * seq_len + s0 + g * lanes, lanes)],
                    fsem).wait()
            return carry

        def fix_batch(b, carry):
            return lax.fori_loop(
                0, per_w // lanes, functools.partial(fix_group, b), carry)

        lax.fori_loop(0, bsz, fix_batch, 0)

    return k


def kernel(input, weights):
    bsz, seq_len = input.shape
    emb = weights.shape[1]
    pos = jnp.arange(1, seq_len + 1, dtype=jnp.int32)
    out = _build(bsz, seq_len, emb)(input.reshape(-1), weights, pos)
    return out.reshape(bsz, seq_len, emb)


# speedup vs baseline: 3.3247x; 1.0198x over previous
import functools

import jax
import jax.numpy as jnp
from jax import lax
from jax.experimental import pallas as pl
from jax.experimental.pallas import tpu as pltpu
from jax.experimental.pallas import tpu_sc as plsc

PADDING_IDX = 0
CHUNK = 32
NBUF = 2


@functools.lru_cache(maxsize=None)
def _build(bsz, seq_len, emb):
    info = plsc.get_sparse_core_info()
    nc, ns, lanes = info.num_cores, info.num_subcores, info.num_lanes
    nw = nc * ns
    ntok = bsz * seq_len
    per_w = seq_len // nw
    assert seq_len % nw == 0 and per_w % CHUNK == 0 and CHUNK % lanes == 0
    nchunks = per_w // CHUNK
    assert nchunks % NBUF == 0
    ngroups = nchunks // NBUF
    mesh = plsc.VectorSubcoreMesh(core_axis_name="c", subcore_axis_name="s")

    scratch = [
        pltpu.VMEM((bsz * per_w,), jnp.int32),
        pltpu.VMEM((per_w,), jnp.int32),
        pltpu.VMEM((lanes, emb), jnp.float32),
    ]
    scratch += [pltpu.VMEM((CHUNK, emb), jnp.float32) for _ in range(NBUF)]
    scratch += [pltpu.SemaphoreType.DMA for _ in range(2 * NBUF + 1)]

    @functools.partial(
        pl.kernel,
        mesh=mesh,
        out_type=jax.ShapeDtypeStruct((ntok, emb), jnp.float32),
        scratch_types=scratch,
    )
    def k(inp_hbm, w_hbm, pos_hbm, out_hbm, tok_v, ridx_v, frows_v, *rest):
        rows = rest[:NBUF]
        gsems = rest[NBUF:2 * NBUF]
        ssems = rest[2 * NBUF:3 * NBUF]
        fsem = rest[3 * NBUF]
        wid = lax.axis_index("s") * nc + lax.axis_index("c")
        s0 = wid * per_w

        for b in range(bsz):
            pltpu.sync_copy(inp_hbm.at[pl.ds(b * seq_len + s0, per_w)],
                            tok_v.at[pl.ds(b * per_w, per_w)])
        pltpu.sync_copy(pos_hbm.at[pl.ds(s0, per_w)], ridx_v)

        lane_iota = lax.iota(jnp.int32, lanes)

        def start_gather(c, buf, gsem):
            t0 = pl.multiple_of(c * CHUNK, CHUNK)
            pltpu.async_copy(w_hbm.at[ridx_v.at[pl.ds(t0, CHUNK)]], buf, gsem)

        def wait_gather(buf, gsem):
            pltpu.make_async_copy(w_hbm.at[pl.ds(0, CHUNK)], buf, gsem).wait()

        def start_scatters(c, buf, ssem):
            for b in range(bsz):
                t0 = b * seq_len + s0 + c * CHUNK
                pltpu.async_copy(buf, out_hbm.at[pl.ds(t0, CHUNK)], ssem)

        def wait_scatters(buf, ssem):
            for _ in range(bsz):
                pltpu.make_async_copy(
                    buf, out_hbm.at[pl.ds(s0, CHUNK)], ssem).wait()

        for j in range(NBUF):
            start_gather(j, rows[j], gsems[j])

        def body(p, carry):
            for j in range(NBUF):
                c = NBUF * p + j
                wait_gather(rows[j], gsems[j])
                start_scatters(c, rows[j], ssems[j])
            @pl.when(p + 1 < ngroups)
            def _():
                for j in range(NBUF):
                    wait_scatters(rows[j], ssems[j])
                    start_gather(NBUF * (p + 1) + j, rows[j], gsems[j])
            return carry

        lax.fori_loop(0, ngroups, body, 0)
        for j in range(NBUF):
            wait_scatters(rows[j], ssems[j])

        def lane_min(v):
            for s in (1, 2, 4, 8):
                perm = jnp.bitwise_xor(lane_iota, s)
                shuf = lax.gather(
                    v, perm[:, None],
                    lax.GatherDimensionNumbers(
                        offset_dims=(), collapsed_slice_dims=(0,),
                        start_index_map=(0,)),
                    (1,), mode=lax.GatherScatterMode.PROMISE_IN_BOUNDS)
                v = jnp.minimum(v, shuf)
            return v

        def fix_group(g, carry):
            ts = [tok_v[pl.ds(b * per_w + g * lanes, lanes)]
                  for b in range(bsz)]
            comb = ts[0]
            for t in ts[1:]:
                comb = jnp.minimum(comb, t)
            trows = lane_iota + (s0 + g * lanes + 1)
            @pl.when(lane_min(comb)[0] == PADDING_IDX)
            def _():
                for b in range(bsz):
                    mask = ts[b] == PADDING_IDX
                    @pl.when(lane_min(ts[b])[0] == PADDING_IDX)
                    def _(b=b, mask=mask):
                        fidx = jnp.where(mask, PADDING_IDX, trows)
                        pltpu.async_copy(
                            w_hbm.at[fidx], frows_v, fsem).wait()
                        pltpu.async_copy(
                            frows_v,
                            out_hbm.at[pl.ds(
                                b * seq_len + s0 + g * lanes, lanes)],
                            fsem).wait()
            return carry

        lax.fori_loop(0, per_w // lanes, fix_group, 0)

    return k


def kernel(input, weights):
    bsz, seq_len = input.shape
    emb = weights.shape[1]
    pos = jnp.arange(1, seq_len + 1, dtype=jnp.int32)
    out = _build(bsz, seq_len, emb)(input.reshape(-1), weights, pos)
    return out.reshape(bsz, seq_len, emb)
